# baseline (speedup 1.0000x reference)
import functools

import jax
import jax.numpy as jnp
from jax import lax
from jax.experimental import pallas as pl
from jax.experimental.pallas import tpu as pltpu
from jax.experimental.pallas import tpu_sc as plsc

NC = 2
NS = 16
LANE = 128
G_OUT = 128
PK = 16
LW = 256
BNP = 128
EB = 512
EBD = 1024


def _mesh():
    return plsc.VectorSubcoreMesh(
        core_axis_name="c", subcore_axis_name="s", num_cores=NC, num_subcores=NS
    )


def _sizes(E, N):
    erows = -(-E // LANE)
    rows_pw = -(-erows // (NC * NS * 16)) * 16
    epad = rows_pw * NC * NS * LANE
    npad = -(-N // (NS * LANE)) * (NS * LANE)
    return epad, npad


def _sc_degree(ei, padb, ones_b, zeros_lf, npad, epad):
    nreal = ei.shape[1]
    nb = (epad // EBD) // (NC * NS)
    npt = npad // NS
    nz = npt // LANE

    @functools.partial(
        pl.kernel,
        mesh=_mesh(),
        out_type=jax.ShapeDtypeStruct((NC, npad, 8), jnp.float32),
        compiler_params=pltpu.CompilerParams(use_tc_tiling_on_sc=False),
        scratch_types=[
            pltpu.VMEM((4, EBD), jnp.int32),
            pltpu.VMEM((EBD, 8), jnp.float32),
            pltpu.VMEM((LANE, 8), jnp.float32),
            pltpu.VMEM_SHARED((npad, 8), jnp.float32),
            pltpu.SemaphoreType.DMA((4,)),
            pltpu.SemaphoreType.DMA((3,)),
        ],
    )
    def k(ei_hbm, pad_hbm, one_hbm, zero_hbm, out_hbm, dstv, onev, zerov, acc, isem, ssem):
        c = lax.axis_index("c")
        sid = lax.axis_index("s")
        wid = sid * NC + c
        pltpu.sync_copy(one_hbm, onev)
        pltpu.sync_copy(zero_hbm, zerov)
        r0 = sid * npt

        def zc(j, carry):
            pltpu.sync_copy(zerov, acc.at[pl.ds(r0 + j * LANE, LANE)])
            return carry

        lax.fori_loop(0, nz, zc, 0)
        plsc.subcore_barrier()
        rb0 = wid * nb

        def idx_start(blk, slot):
            g = rb0 + blk

            @pl.when(g < nreal)
            def _():
                pltpu.async_copy(
                    ei_hbm.at[1, pl.ds(g, 1)], dstv.at[pl.ds(slot, 1)], isem.at[slot]
                )

            @pl.when(g >= nreal)
            def _():
                pltpu.async_copy(
                    pad_hbm.at[1, pl.ds(g - nreal, 1)],
                    dstv.at[pl.ds(slot, 1)],
                    isem.at[slot],
                )

        def idx_wait(blk, slot):
            g = rb0 + blk

            @pl.when(g < nreal)
            def _():
                pltpu.make_async_copy(
                    ei_hbm.at[1, pl.ds(g, 1)], dstv.at[pl.ds(slot, 1)], isem.at[slot]
                ).wait()

            @pl.when(g >= nreal)
            def _():
                pltpu.make_async_copy(
                    pad_hbm.at[1, pl.ds(g - nreal, 1)],
                    dstv.at[pl.ds(slot, 1)],
                    isem.at[slot],
                ).wait()

        def sc_start(slot, sslot):
            pltpu.async_copy(onev, acc.at[dstv.at[slot]], ssem.at[sslot], add=True)

        def sc_wait(slot, sslot):
            pltpu.make_async_copy(
                onev, acc.at[dstv.at[slot]], ssem.at[sslot]
            ).wait()

        idx_start(0, 0)
        if nb > 1:
            idx_start(1, 1)
        idx_wait(0, 0)

        def body(blk, carry):
            cur = lax.rem(blk, 4)

            @pl.when(blk > 1)
            def _():
                sc_wait(lax.rem(blk + 2, 4), lax.rem(blk + 1, 3))

            @pl.when(blk + 1 < nb)
            def _():
                idx_wait(blk + 1, lax.rem(blk + 1, 4))

            sc_start(cur, lax.rem(blk, 3))

            @pl.when(blk + 2 < nb)
            def _():
                idx_start(blk + 2, lax.rem(blk + 2, 4))

            return carry

        lax.fori_loop(0, nb, body, 0)
        sc_wait(lax.rem(nb - 1, 4), lax.rem(nb - 1, 3))
        if nb > 1:
            sc_wait((nb - 2) % 4, (nb - 2) % 3)
        plsc.subcore_barrier()
        pltpu.sync_copy(acc.at[pl.ds(r0, npt)], out_hbm.at[c, pl.ds(r0, npt)])

    return k(ei, padb, ones_b, zeros_lf)


def _sc_aggregate(ei, padb, table, zeros_lf, npad, epad):
    nreal = ei.shape[1]
    nb = (epad // EB) // (NC * NS)
    npt = npad // NS
    nz = npt // LANE

    @functools.partial(
        pl.kernel,
        mesh=_mesh(),
        out_type=jax.ShapeDtypeStruct((NC, npad, 16), jnp.float32),
        compiler_params=pltpu.CompilerParams(use_tc_tiling_on_sc=False),
        scratch_types=[
            pltpu.VMEM((4, EB), jnp.int32),
            pltpu.VMEM((4, EB), jnp.int32),
            pltpu.VMEM((3, EB, 16), jnp.float32),
            pltpu.VMEM((LANE, 16), jnp.float32),
            pltpu.VMEM_SHARED((npad, 16), jnp.float32),
            pltpu.SemaphoreType.DMA((4,)),
            pltpu.SemaphoreType.DMA((4,)),
            pltpu.SemaphoreType.DMA((3,)),
        ],
    )
    def k(
        ei_hbm, pad_hbm, tab_hbm, zero_hbm, out_hbm,
        srcv, dstv, rows, zbuf, acc, gsem, isem, ssem,
    ):
        c = lax.axis_index("c")
        sid = lax.axis_index("s")
        wid = sid * NC + c
        pltpu.sync_copy(zero_hbm, zbuf)
        r0 = sid * npt

        def zc(j, carry):
            pltpu.sync_copy(zbuf, acc.at[pl.ds(r0 + j * LANE, LANE)])
            return carry

        lax.fori_loop(0, nz, zc, 0)
        plsc.subcore_barrier()
        rb0 = wid * nb

        def idx_start(blk, slot):
            g = rb0 + blk

            @pl.when(g < nreal)
            def _():
                pltpu.async_copy(
                    ei_hbm.at[0, pl.ds(g, 1)], srcv.at[pl.ds(slot, 1)], isem.at[slot]
                )
                pltpu.async_copy(
                    ei_hbm.at[1, pl.ds(g, 1)], dstv.at[pl.ds(slot, 1)], isem.at[slot]
                )

            @pl.when(g >= nreal)
            def _():
                pltpu.async_copy(
                    pad_hbm.at[0, pl.ds(g - nreal, 1)],
                    srcv.at[pl.ds(slot, 1)],
                    isem.at[slot],
                )
                pltpu.async_copy(
                    pad_hbm.at[1, pl.ds(g - nreal, 1)],
                    dstv.at[pl.ds(slot, 1)],
                    isem.at[slot],
                )

        def idx_wait(blk, slot):
            g = rb0 + blk

            @pl.when(g < nreal)
            def _():
                pltpu.make_async_copy(
                    ei_hbm.at[0, pl.ds(g, 1)], srcv.at[pl.ds(slot, 1)], isem.at[slot]
                ).wait()
                pltpu.make_async_copy(
                    ei_hbm.at[1, pl.ds(g, 1)], dstv.at[pl.ds(slot, 1)], isem.at[slot]
                ).wait()

            @pl.when(g >= nreal)
            def _():
                pltpu.make_async_copy(
                    pad_hbm.at[0, pl.ds(g - nreal, 1)],
                    srcv.at[pl.ds(slot, 1)],
                    isem.at[slot],
                ).wait()
                pltpu.make_async_copy(
                    pad_hbm.at[1, pl.ds(g - nreal, 1)],
                    dstv.at[pl.ds(slot, 1)],
                    isem.at[slot],
                ).wait()

        def gather_start(islot, rslot):
            pltpu.async_copy(tab_hbm.at[srcv.at[islot]], rows.at[rslot], gsem.at[islot])

        def gather_wait(islot, rslot):
            pltpu.make_async_copy(
                tab_hbm.at[srcv.at[islot]], rows.at[rslot], gsem.at[islot]
            ).wait()

        def sc_start(islot, rslot, sslot):
            pltpu.async_copy(
                rows.at[rslot], acc.at[dstv.at[islot]], ssem.at[sslot], add=True
            )

        def sc_wait(islot, rslot, sslot):
            pltpu.make_async_copy(
                rows.at[rslot], acc.at[dstv.at[islot]], ssem.at[sslot]
            ).wait()

        idx_start(0, 0)
        idx_wait(0, 0)
        gather_start(0, 0)
        if nb > 1:
            idx_start(1, 1)

        def body(blk, carry):
            i_cur = lax.rem(blk, 4)
            i_nxt = lax.rem(blk + 1, 4)
            r_cur = lax.rem(blk, 3)
            r_nxt = lax.rem(blk + 1, 3)

            @pl.when(blk > 1)
            def _():
                sc_wait(lax.rem(blk + 2, 4), r_nxt, lax.rem(blk + 1, 3))

            @pl.when(blk + 1 < nb)
            def _():
                idx_wait(blk + 1, i_nxt)
                gather_start(i_nxt, r_nxt)

            gather_wait(i_cur, r_cur)
            sc_start(i_cur, r_cur, r_cur)

            @pl.when(blk + 2 < nb)
            def _():
                idx_start(blk + 2, lax.rem(blk + 2, 4))

            return carry

        lax.fori_loop(0, nb, body, 0)
        sc_wait(lax.rem(nb - 1, 4), lax.rem(nb - 1, 3), lax.rem(nb - 1, 3))
        if nb > 1:
            sc_wait((nb - 2) % 4, (nb - 2) % 3, (nb - 2) % 3)
        plsc.subcore_barrier()
        pltpu.sync_copy(acc.at[pl.ds(r0, npt)], out_hbm.at[c, pl.ds(r0, npt)])

    return k(ei, padb, table, zeros_lf)


def _tc_stage_a(dpp, xp, bdw1, pe, npad):
    npk = npad // PK
    grid = npk // BNP

    def body(dp_ref, x_ref, w_ref, pe_ref, s_ref, t_ref):
        s8 = lax.rsqrt(dp_ref[0] + dp_ref[1] + 1.0)
        s = jnp.dot(s8, pe_ref[...], preferred_element_type=jnp.float32)
        xw = jnp.dot(x_ref[...], w_ref[...], preferred_element_type=jnp.float32)
        s_ref[...] = s
        t_ref[...] = xw * s

    return pl.pallas_call(
        body,
        grid=(grid,),
        in_specs=[
            pl.BlockSpec((NC, BNP, LANE), lambda i: (0, i, 0)),
            pl.BlockSpec((BNP, PK * 4), lambda i: (i, 0)),
            pl.BlockSpec((PK * 4, LW), lambda i: (0, 0)),
            pl.BlockSpec((LANE, LW), lambda i: (0, 0)),
        ],
        out_specs=[
            pl.BlockSpec((BNP, LW), lambda i: (i, 0)),
            pl.BlockSpec((BNP, LW), lambda i: (i, 0)),
        ],
        out_shape=[
            jax.ShapeDtypeStruct((npk, LW), jnp.float32),
            jax.ShapeDtypeStruct((npk, LW), jnp.float32),
        ],
    )(dpp, xp, bdw1, pe)


def _tc_stage_b(g1p, t1p, sp, bdw2, b1t, npad):
    npk = npad // PK
    grid = npk // BNP

    def body(g_ref, t_ref, s_ref, w_ref, b_ref, t2_ref):
        s = s_ref[...]
        h1 = jnp.maximum(s * (g_ref[0] + g_ref[1] + t_ref[...]) + b_ref[...], 0.0)
        hw = jnp.dot(h1, w_ref[...], preferred_element_type=jnp.float32)
        t2_ref[...] = hw * s

    return pl.pallas_call(
        body,
        grid=(grid,),
        in_specs=[
            pl.BlockSpec((NC, BNP, LW), lambda i: (0, i, 0)),
            pl.BlockSpec((BNP, LW), lambda i: (i, 0)),
            pl.BlockSpec((BNP, LW), lambda i: (i, 0)),
            pl.BlockSpec((LW, LW), lambda i: (0, 0)),
            pl.BlockSpec((1, LW), lambda i: (0, 0)),
        ],
        out_specs=pl.BlockSpec((BNP, LW), lambda i: (i, 0)),
        out_shape=jax.ShapeDtypeStruct((npk, LW), jnp.float32),
    )(g1p, t1p, sp, bdw2, b1t)


def _tc_stage_c(g2p, t2p, sp, b2t, batch2d, fc1_w, fc1_br, fc2_w, fc2_br, npad):
    npk = npad // PK
    grid = npk // BNP

    def body(g_ref, t_ref, s_ref, b_ref, bt_ref, f1w, f1b, f2w, f2b, out_ref, pooled):
        i = pl.program_id(0)

        @pl.when(i == 0)
        def _init():
            pooled[...] = jnp.zeros_like(pooled)

        h2 = jnp.maximum(
            s_ref[...] * (g_ref[0] + g_ref[1] + t_ref[...]) + b_ref[...], 0.0
        )
        iota = lax.broadcasted_iota(jnp.int32, (BNP, G_OUT), 1)
        acc = jnp.zeros((G_OUT, 16), jnp.float32)
        for kk in range(PK):
            oh = (bt_ref[:, kk : kk + 1] == iota).astype(jnp.float32)
            acc += lax.dot_general(
                oh,
                h2[:, kk * 16 : (kk + 1) * 16],
                (((0,), (0,)), ((), ())),
                preferred_element_type=jnp.float32,
            )
        pooled[...] += acc

        @pl.when(i == pl.num_programs(0) - 1)
        def _final():
            p = jnp.maximum(
                jnp.dot(pooled[...], f1w[...], preferred_element_type=jnp.float32)
                + f1b[...],
                0.0,
            )
            out_ref[...] = (
                jnp.dot(p, f2w[...], preferred_element_type=jnp.float32) + f2b[...]
            )

    return pl.pallas_call(
        body,
        grid=(grid,),
        in_specs=[
            pl.BlockSpec((NC, BNP, LW), lambda i: (0, i, 0)),
            pl.BlockSpec((BNP, LW), lambda i: (i, 0)),
            pl.BlockSpec((BNP, LW), lambda i: (i, 0)),
            pl.BlockSpec((1, LW), lambda i: (0, 0)),
            pl.BlockSpec((BNP, PK), lambda i: (i, 0)),
            pl.BlockSpec((16, 16), lambda i: (0, 0)),
            pl.BlockSpec((1, 16), lambda i: (0, 0)),
            pl.BlockSpec((16, 1), lambda i: (0, 0)),
            pl.BlockSpec((1, 1), lambda i: (0, 0)),
        ],
        out_specs=pl.BlockSpec((G_OUT, 1), lambda i: (0, 0)),
        out_shape=jax.ShapeDtypeStruct((G_OUT, 1), jnp.float32),
        scratch_shapes=[pltpu.VMEM((G_OUT, 16), jnp.float32)],
    )(g2p, t2p, sp, b2t, batch2d, fc1_w, fc1_br, fc2_w, fc2_br)


def kernel(x, edge_index, batch, W1, b1, W2, b2, fc1_w, fc1_b, fc2_w, fc2_b):
    N, F_IN = x.shape
    E = edge_index.shape[1]
    epad, npad = _sizes(E, N)
    pad = epad - E
    npk = npad // PK

    ei_a = edge_index.reshape(2, E // EB, EB)
    ei_d = edge_index.reshape(2, E // EBD, EBD)
    pad_i = jnp.arange(pad, dtype=jnp.int32)
    padb = jnp.stack([pad_i % N, N + pad_i % (npad - N)])
    pad_a = padb.reshape(2, pad // EB, EB)
    pad_d = padb.reshape(2, pad // EBD, EBD)

    xp = jnp.concatenate([x, jnp.zeros((npad - N, F_IN), x.dtype)]).reshape(
        npk, PK * F_IN
    )
    batch2d = jnp.concatenate(
        [batch, jnp.full((npad - N,), 2**30, jnp.int32)]
    ).reshape(npk, PK)
    ones_b = jnp.ones((EBD, 8), jnp.float32)
    zeros_lf = jnp.zeros((LANE, 16), jnp.float32)
    zeros_l8 = jnp.zeros((LANE, 8), jnp.float32)

    eye = jnp.eye(PK, dtype=jnp.float32)
    bdw1 = jnp.kron(eye, W1)
    bdw2 = jnp.kron(eye, W2)
    b1t = jnp.tile(b1, PK).reshape(1, PK * 16)
    b2t = jnp.tile(b2, PK).reshape(1, PK * 16)
    pe = jnp.zeros((LANE, LW), jnp.float32)
    m = jnp.arange(PK)
    pe = pe.at[8 * m[:, None], 16 * m[:, None] + jnp.arange(16)[None, :]].set(1.0)

    dp = _sc_degree(ei_d, pad_d, ones_b, zeros_l8, npad, epad)
    dpp = dp.reshape(NC, npk, LANE)
    sp, t1p = _tc_stage_a(dpp, xp, bdw1, pe, npad)
    g1 = _sc_aggregate(ei_a, pad_a, t1p.reshape(npad, 16), zeros_lf, npad, epad)
    t2p = _tc_stage_b(g1.reshape(NC, npk, LW), t1p, sp, bdw2, b1t, npad)
    g2 = _sc_aggregate(ei_a, pad_a, t2p.reshape(npad, 16), zeros_lf, npad, epad)
    return _tc_stage_c(
        g2.reshape(NC, npk, LW),
        t2p,
        sp,
        b2t,
        batch2d,
        fc1_w,
        fc1_b.reshape(1, 16),
        fc2_w,
        fc2_b.reshape(1, 1),
        npad,
    )


# speedup vs baseline: 1.1354x; 1.1354x over previous
import functools

import jax
import jax.numpy as jnp
from jax import lax
from jax.experimental import pallas as pl
from jax.experimental.pallas import tpu as pltpu
from jax.experimental.pallas import tpu_sc as plsc

NC = 2
NS = 16
LANE = 128
G_OUT = 128
PK = 8
LW = 128
BNP = 784
EB = 512
EBD = 1024


def _mesh():
    return plsc.VectorSubcoreMesh(
        core_axis_name="c", subcore_axis_name="s", num_cores=NC, num_subcores=NS
    )


def _sizes(E, N):
    erows = -(-E // LANE)
    rows_pw = -(-erows // (NC * NS * 16)) * 16
    epad = rows_pw * NC * NS * LANE
    npad = -(-N // (NS * LANE)) * (NS * LANE)
    return epad, npad


def _sc_degree(ei, padb, ones_b, zeros_lf, npad, epad):
    nreal = ei.shape[1]
    nb = (epad // EBD) // (NC * NS)
    npt = npad // NS
    nz = npt // LANE

    @functools.partial(
        pl.kernel,
        mesh=_mesh(),
        out_type=jax.ShapeDtypeStruct((NC, npad, 16), jnp.float32),
        compiler_params=pltpu.CompilerParams(use_tc_tiling_on_sc=False),
        scratch_types=[
            pltpu.VMEM((4, EBD), jnp.int32),
            pltpu.VMEM((EBD, 16), jnp.float32),
            pltpu.VMEM((LANE, 16), jnp.float32),
            pltpu.VMEM_SHARED((npad, 16), jnp.float32),
            pltpu.SemaphoreType.DMA((4,)),
            pltpu.SemaphoreType.DMA((3,)),
        ],
    )
    def k(ei_hbm, pad_hbm, one_hbm, zero_hbm, out_hbm, dstv, onev, zerov, acc, isem, ssem):
        c = lax.axis_index("c")
        sid = lax.axis_index("s")
        wid = sid * NC + c
        pltpu.sync_copy(one_hbm, onev)
        pltpu.sync_copy(zero_hbm, zerov)
        r0 = sid * npt

        def zc(j, carry):
            pltpu.sync_copy(zerov, acc.at[pl.ds(r0 + j * LANE, LANE)])
            return carry

        lax.fori_loop(0, nz, zc, 0)
        plsc.subcore_barrier()
        rb0 = wid * nb

        def idx_start(blk, slot):
            g = rb0 + blk

            @pl.when(g < nreal)
            def _():
                pltpu.async_copy(
                    ei_hbm.at[1, pl.ds(g, 1)], dstv.at[pl.ds(slot, 1)], isem.at[slot]
                )

            @pl.when(g >= nreal)
            def _():
                pltpu.async_copy(
                    pad_hbm.at[1, pl.ds(g - nreal, 1)],
                    dstv.at[pl.ds(slot, 1)],
                    isem.at[slot],
                )

        def idx_wait(blk, slot):
            g = rb0 + blk

            @pl.when(g < nreal)
            def _():
                pltpu.make_async_copy(
                    ei_hbm.at[1, pl.ds(g, 1)], dstv.at[pl.ds(slot, 1)], isem.at[slot]
                ).wait()

            @pl.when(g >= nreal)
            def _():
                pltpu.make_async_copy(
                    pad_hbm.at[1, pl.ds(g - nreal, 1)],
                    dstv.at[pl.ds(slot, 1)],
                    isem.at[slot],
                ).wait()

        def sc_start(slot, sslot):
            pltpu.async_copy(onev, acc.at[dstv.at[slot]], ssem.at[sslot], add=True)

        def sc_wait(slot, sslot):
            pltpu.make_async_copy(
                onev, acc.at[dstv.at[slot]], ssem.at[sslot]
            ).wait()

        idx_start(0, 0)
        if nb > 1:
            idx_start(1, 1)
        idx_wait(0, 0)

        def body(blk, carry):
            cur = lax.rem(blk, 4)

            @pl.when(blk > 1)
            def _():
                sc_wait(lax.rem(blk + 2, 4), lax.rem(blk + 1, 3))

            @pl.when(blk + 1 < nb)
            def _():
                idx_wait(blk + 1, lax.rem(blk + 1, 4))

            sc_start(cur, lax.rem(blk, 3))

            @pl.when(blk + 2 < nb)
            def _():
                idx_start(blk + 2, lax.rem(blk + 2, 4))

            return carry

        lax.fori_loop(0, nb, body, 0)
        sc_wait(lax.rem(nb - 1, 4), lax.rem(nb - 1, 3))
        if nb > 1:
            sc_wait((nb - 2) % 4, (nb - 2) % 3)
        plsc.subcore_barrier()
        pltpu.sync_copy(acc.at[pl.ds(r0, npt)], out_hbm.at[c, pl.ds(r0, npt)])

    return k(ei, padb, ones_b, zeros_lf)


def _sc_aggregate(ei, padb, table, zeros_lf, npad, epad):
    nreal = ei.shape[1]
    nb = (epad // EB) // (NC * NS)
    npt = npad // NS
    nz = npt // LANE

    @functools.partial(
        pl.kernel,
        mesh=_mesh(),
        out_type=jax.ShapeDtypeStruct((NC, npad, 16), jnp.float32),
        compiler_params=pltpu.CompilerParams(use_tc_tiling_on_sc=False),
        scratch_types=[
            pltpu.VMEM((4, EB), jnp.int32),
            pltpu.VMEM((4, EB), jnp.int32),
            pltpu.VMEM((3, EB, 16), jnp.float32),
            pltpu.VMEM((LANE, 16), jnp.float32),
            pltpu.VMEM_SHARED((npad, 16), jnp.float32),
            pltpu.SemaphoreType.DMA((4,)),
            pltpu.SemaphoreType.DMA((4,)),
            pltpu.SemaphoreType.DMA((3,)),
        ],
    )
    def k(
        ei_hbm, pad_hbm, tab_hbm, zero_hbm, out_hbm,
        srcv, dstv, rows, zbuf, acc, gsem, isem, ssem,
    ):
        c = lax.axis_index("c")
        sid = lax.axis_index("s")
        wid = sid * NC + c
        pltpu.sync_copy(zero_hbm, zbuf)
        r0 = sid * npt

        def zc(j, carry):
            pltpu.sync_copy(zbuf, acc.at[pl.ds(r0 + j * LANE, LANE)])
            return carry

        lax.fori_loop(0, nz, zc, 0)
        plsc.subcore_barrier()
        rb0 = wid * nb

        def idx_start(blk, slot):
            g = rb0 + blk

            @pl.when(g < nreal)
            def _():
                pltpu.async_copy(
                    ei_hbm.at[0, pl.ds(g, 1)], srcv.at[pl.ds(slot, 1)], isem.at[slot]
                )
                pltpu.async_copy(
                    ei_hbm.at[1, pl.ds(g, 1)], dstv.at[pl.ds(slot, 1)], isem.at[slot]
                )

            @pl.when(g >= nreal)
            def _():
                pltpu.async_copy(
                    pad_hbm.at[0, pl.ds(g - nreal, 1)],
                    srcv.at[pl.ds(slot, 1)],
                    isem.at[slot],
                )
                pltpu.async_copy(
                    pad_hbm.at[1, pl.ds(g - nreal, 1)],
                    dstv.at[pl.ds(slot, 1)],
                    isem.at[slot],
                )

        def idx_wait(blk, slot):
            g = rb0 + blk

            @pl.when(g < nreal)
            def _():
                pltpu.make_async_copy(
                    ei_hbm.at[0, pl.ds(g, 1)], srcv.at[pl.ds(slot, 1)], isem.at[slot]
                ).wait()
                pltpu.make_async_copy(
                    ei_hbm.at[1, pl.ds(g, 1)], dstv.at[pl.ds(slot, 1)], isem.at[slot]
                ).wait()

            @pl.when(g >= nreal)
            def _():
                pltpu.make_async_copy(
                    pad_hbm.at[0, pl.ds(g - nreal, 1)],
                    srcv.at[pl.ds(slot, 1)],
                    isem.at[slot],
                ).wait()
                pltpu.make_async_copy(
                    pad_hbm.at[1, pl.ds(g - nreal, 1)],
                    dstv.at[pl.ds(slot, 1)],
                    isem.at[slot],
                ).wait()

        def gather_start(islot, rslot):
            pltpu.async_copy(tab_hbm.at[srcv.at[islot]], rows.at[rslot], gsem.at[islot])

        def gather_wait(islot, rslot):
            pltpu.make_async_copy(
                tab_hbm.at[srcv.at[islot]], rows.at[rslot], gsem.at[islot]
            ).wait()

        def sc_start(islot, rslot, sslot):
            pltpu.async_copy(
                rows.at[rslot], acc.at[dstv.at[islot]], ssem.at[sslot], add=True
            )

        def sc_wait(islot, rslot, sslot):
            pltpu.make_async_copy(
                rows.at[rslot], acc.at[dstv.at[islot]], ssem.at[sslot]
            ).wait()

        idx_start(0, 0)
        idx_wait(0, 0)
        gather_start(0, 0)
        if nb > 1:
            idx_start(1, 1)

        def body(blk, carry):
            i_cur = lax.rem(blk, 4)
            i_nxt = lax.rem(blk + 1, 4)
            r_cur = lax.rem(blk, 3)
            r_nxt = lax.rem(blk + 1, 3)

            @pl.when(blk > 1)
            def _():
                sc_wait(lax.rem(blk + 2, 4), r_nxt, lax.rem(blk + 1, 3))

            @pl.when(blk + 1 < nb)
            def _():
                idx_wait(blk + 1, i_nxt)
                gather_start(i_nxt, r_nxt)

            gather_wait(i_cur, r_cur)
            sc_start(i_cur, r_cur, r_cur)

            @pl.when(blk + 2 < nb)
            def _():
                idx_start(blk + 2, lax.rem(blk + 2, 4))

            return carry

        lax.fori_loop(0, nb, body, 0)
        sc_wait(lax.rem(nb - 1, 4), lax.rem(nb - 1, 3), lax.rem(nb - 1, 3))
        if nb > 1:
            sc_wait((nb - 2) % 4, (nb - 2) % 3, (nb - 2) % 3)
        plsc.subcore_barrier()
        pltpu.sync_copy(acc.at[pl.ds(r0, npt)], out_hbm.at[c, pl.ds(r0, npt)])

    return k(ei, padb, table, zeros_lf)


def _tc_xw(xp, bdw1, npad):
    npk = npad // PK
    grid = npk // BNP

    def body(x_ref, w_ref, o_ref):
        o_ref[...] = jnp.dot(
            x_ref[...], w_ref[...], preferred_element_type=jnp.float32
        )

    return pl.pallas_call(
        body,
        grid=(grid,),
        in_specs=[
            pl.BlockSpec((BNP, PK * 4), lambda i: (i, 0)),
            pl.BlockSpec((PK * 4, LW), lambda i: (0, 0)),
        ],
        out_specs=pl.BlockSpec((BNP, LW), lambda i: (i, 0)),
        out_shape=jax.ShapeDtypeStruct((npk, LW), jnp.float32),
    )(xp, bdw1)


def _tc_stage_a(dpp, xwp, npad):
    npk = npad // PK
    grid = npk // BNP

    def body(dp_ref, xw_ref, s_ref, t_ref):
        s = lax.rsqrt(dp_ref[0] + dp_ref[1] + 1.0)
        s_ref[...] = s
        t_ref[...] = xw_ref[...] * s

    return pl.pallas_call(
        body,
        grid=(grid,),
        in_specs=[
            pl.BlockSpec((NC, BNP, LANE), lambda i: (0, i, 0)),
            pl.BlockSpec((BNP, LW), lambda i: (i, 0)),
        ],
        out_specs=[
            pl.BlockSpec((BNP, LW), lambda i: (i, 0)),
            pl.BlockSpec((BNP, LW), lambda i: (i, 0)),
        ],
        out_shape=[
            jax.ShapeDtypeStruct((npk, LW), jnp.float32),
            jax.ShapeDtypeStruct((npk, LW), jnp.float32),
        ],
    )(dpp, xwp)


def _tc_stage_b(g1p, t1p, sp, bdw2, b1t, npad):
    npk = npad // PK
    grid = npk // BNP

    def body(g_ref, t_ref, s_ref, w_ref, b_ref, t2_ref):
        s = s_ref[...]
        h1 = jnp.maximum(s * (g_ref[0] + g_ref[1] + t_ref[...]) + b_ref[...], 0.0)
        hw = jnp.dot(h1, w_ref[...], preferred_element_type=jnp.float32)
        t2_ref[...] = hw * s

    return pl.pallas_call(
        body,
        grid=(grid,),
        in_specs=[
            pl.BlockSpec((NC, BNP, LW), lambda i: (0, i, 0)),
            pl.BlockSpec((BNP, LW), lambda i: (i, 0)),
            pl.BlockSpec((BNP, LW), lambda i: (i, 0)),
            pl.BlockSpec((LW, LW), lambda i: (0, 0)),
            pl.BlockSpec((1, LW), lambda i: (0, 0)),
        ],
        out_specs=pl.BlockSpec((BNP, LW), lambda i: (i, 0)),
        out_shape=jax.ShapeDtypeStruct((npk, LW), jnp.float32),
    )(g1p, t1p, sp, bdw2, b1t)


def _tc_stage_c(g2p, t2p, sp, b2t, batch2d, fc1_w, fc1_br, fc2_w, fc2_br, npad):
    npk = npad // PK
    grid = npk // BNP

    def body(g_ref, t_ref, s_ref, b_ref, bt_ref, f1w, f1b, f2w, f2b, out_ref, pooled):
        i = pl.program_id(0)

        @pl.when(i == 0)
        def _init():
            pooled[...] = jnp.zeros_like(pooled)

        h2 = jnp.maximum(
            s_ref[...] * (g_ref[0] + g_ref[1] + t_ref[...]) + b_ref[...], 0.0
        )
        iota = lax.broadcasted_iota(jnp.int32, (BNP, G_OUT), 1)
        acc = jnp.zeros((G_OUT, 16), jnp.float32)
        for kk in range(PK):
            oh = (bt_ref[:, kk : kk + 1] == iota).astype(jnp.float32)
            acc += lax.dot_general(
                oh,
                h2[:, kk * 16 : (kk + 1) * 16],
                (((0,), (0,)), ((), ())),
                preferred_element_type=jnp.float32,
            )
        pooled[...] += acc

        @pl.when(i == pl.num_programs(0) - 1)
        def _final():
            p = jnp.maximum(
                jnp.dot(pooled[...], f1w[...], preferred_element_type=jnp.float32)
                + f1b[...],
                0.0,
            )
            out_ref[...] = (
                jnp.dot(p, f2w[...], preferred_element_type=jnp.float32) + f2b[...]
            )

    return pl.pallas_call(
        body,
        grid=(grid,),
        in_specs=[
            pl.BlockSpec((NC, BNP, LW), lambda i: (0, i, 0)),
            pl.BlockSpec((BNP, LW), lambda i: (i, 0)),
            pl.BlockSpec((BNP, LW), lambda i: (i, 0)),
            pl.BlockSpec((1, LW), lambda i: (0, 0)),
            pl.BlockSpec((BNP, PK), lambda i: (i, 0)),
            pl.BlockSpec((16, 16), lambda i: (0, 0)),
            pl.BlockSpec((1, 16), lambda i: (0, 0)),
            pl.BlockSpec((16, 1), lambda i: (0, 0)),
            pl.BlockSpec((1, 1), lambda i: (0, 0)),
        ],
        out_specs=pl.BlockSpec((G_OUT, 1), lambda i: (0, 0)),
        out_shape=jax.ShapeDtypeStruct((G_OUT, 1), jnp.float32),
        scratch_shapes=[pltpu.VMEM((G_OUT, 16), jnp.float32)],
    )(g2p, t2p, sp, b2t, batch2d, fc1_w, fc1_br, fc2_w, fc2_br)


def kernel(x, edge_index, batch, W1, b1, W2, b2, fc1_w, fc1_b, fc2_w, fc2_b):
    N, F_IN = x.shape
    E = edge_index.shape[1]
    epad, npad = _sizes(E, N)
    pad = epad - E
    npk = npad // PK

    ei_a = edge_index.reshape(2, E // EB, EB)
    ei_d = edge_index.reshape(2, E // EBD, EBD)
    pad_i = jnp.arange(pad, dtype=jnp.int32)
    padb = jnp.stack([pad_i % N, N + pad_i % (npad - N)])
    pad_a = padb.reshape(2, pad // EB, EB)
    pad_d = padb.reshape(2, pad // EBD, EBD)

    xp = jnp.concatenate([x, jnp.zeros((npad - N, F_IN), x.dtype)]).reshape(
        npk, PK * F_IN
    )
    batch2d = jnp.concatenate(
        [batch, jnp.full((npad - N,), 2**30, jnp.int32)]
    ).reshape(npk, PK)
    ones_b = jnp.ones((EBD, 16), jnp.float32)
    zeros_lf = jnp.zeros((LANE, 16), jnp.float32)

    eye = jnp.eye(PK, dtype=jnp.float32)
    bdw1 = jnp.kron(eye, W1)
    bdw2 = jnp.kron(eye, W2)
    b1t = jnp.tile(b1, PK).reshape(1, PK * 16)
    b2t = jnp.tile(b2, PK).reshape(1, PK * 16)

    xwp = _tc_xw(xp, bdw1, npad)
    dp = _sc_degree(ei_d, pad_d, ones_b, zeros_lf, npad, epad)
    dpp = dp.reshape(NC, npk, LANE)
    sp, t1p = _tc_stage_a(dpp, xwp, npad)
    g1 = _sc_aggregate(ei_a, pad_a, t1p.reshape(npad, 16), zeros_lf, npad, epad)
    t2p = _tc_stage_b(g1.reshape(NC, npk, LW), t1p, sp, bdw2, b1t, npad)
    g2 = _sc_aggregate(ei_a, pad_a, t2p.reshape(npad, 16), zeros_lf, npad, epad)
    return _tc_stage_c(
        g2.reshape(NC, npk, LW),
        t2p,
        sp,
        b2t,
        batch2d,
        fc1_w,
        fc1_b.reshape(1, 16),
        fc2_w,
        fc2_b.reshape(1, 1),
        npad,
    )
